# Initial kernel scaffold; baseline (speedup 1.0000x reference)
import jax
import jax.numpy as jnp
from jax.experimental import pallas as pl


def kernel(h_gene, h_disease, edge_index):
    raise NotImplementedError("write your pallas kernel here")


# speedup vs baseline: 3.0425x; 3.0425x over previous
import functools

import jax
import jax.numpy as jnp
from jax import lax
from jax.experimental import pallas as pl
from jax.experimental.pallas import tpu as pltpu
from jax.experimental.pallas import tpu_sc as plsc

N_GENE = 10000
N_DISEASE = 10000
E = 320000
D = 128

NC = 2
NS = 16
NW = NC * NS
PW = E // NW
G = 80
NCHUNK = PW // G


def _normalize_body(g_ref, d_ref, go_ref, do_ref):
    x = g_ref[...]
    go_ref[...] = x / jnp.sqrt(jnp.sum(x * x, axis=1, keepdims=True))
    y = d_ref[...]
    do_ref[...] = y / jnp.sqrt(jnp.sum(y * y, axis=1, keepdims=True))


def _normalize(h_gene, h_disease):
    return pl.pallas_call(
        _normalize_body,
        out_shape=(
            jax.ShapeDtypeStruct((N_GENE, D), jnp.float32),
            jax.ShapeDtypeStruct((N_DISEASE, D), jnp.float32),
        ),
    )(h_gene, h_disease)


@functools.partial(
    pl.kernel,
    mesh=plsc.VectorSubcoreMesh(core_axis_name="c", subcore_axis_name="s",
                                num_cores=NC),
    out_type=jax.ShapeDtypeStruct((E,), jnp.float32),
    compiler_params=pltpu.CompilerParams(needs_layout_passes=False),
    scratch_types=[
        pltpu.VMEM((PW,), jnp.int32),
        pltpu.VMEM((PW,), jnp.int32),
        pltpu.VMEM((G, D), jnp.float32),
        pltpu.VMEM((G, D), jnp.float32),
        pltpu.VMEM((PW,), jnp.float32),
        pltpu.SemaphoreType.DMA,
    ],
)
def _edge_dots(hg_hbm, hd_hbm, src_hbm, dst_hbm, out_hbm,
               src_v, dst_v, u_v, v_v, o_v, sem):
    wid = lax.axis_index("s") * NC + lax.axis_index("c")
    base = pl.multiple_of(wid * PW, 8)

    pltpu.sync_copy(src_hbm.at[pl.ds(base, PW)], src_v)
    pltpu.sync_copy(dst_hbm.at[pl.ds(base, PW)], dst_v)

    def chunk_body(c, _):
        off = pl.multiple_of(c * G, 8)
        pltpu.async_copy(hg_hbm.at[src_v.at[pl.ds(off, G)]], u_v, sem).wait()
        pltpu.async_copy(hd_hbm.at[dst_v.at[pl.ds(off, G)]], v_v, sem).wait()

        for g in range(G // 16):
            rows = lax.iota(jnp.int32, 16) + (g * 16)

            def d_body(d, acc):
                cols = jnp.zeros((16,), jnp.int32) + d
                du = plsc.load_gather(u_v, [rows, cols])
                dv = plsc.load_gather(v_v, [rows, cols])
                return acc + du * dv

            acc = lax.fori_loop(0, D, d_body, jnp.zeros((16,), jnp.float32),
                                unroll=16)
            o_v[pl.ds(off + g * 16, 16)] = acc
        return 0

    lax.fori_loop(0, NCHUNK, chunk_body, 0)
    pltpu.sync_copy(o_v, out_hbm.at[pl.ds(base, PW)])


def kernel(h_gene, h_disease, edge_index):
    gn, dn = _normalize(h_gene, h_disease)
    src = edge_index[0].astype(jnp.int32)
    dst = edge_index[1].astype(jnp.int32)
    out = _edge_dots(gn, dn, src, dst)
    return out.reshape(E, 1)


# speedup vs baseline: 23.0705x; 7.5828x over previous
import functools

import jax
import jax.numpy as jnp
from jax import lax
from jax.experimental import pallas as pl
from jax.experimental.pallas import tpu as pltpu
from jax.experimental.pallas import tpu_sc as plsc

N_GENE = 10000
N_DISEASE = 10000
E = 320000
D = 128

NC = 2
NS = 16
NW = NC * NS
PW = E // NW
G = 80
NCHUNK = PW // G


def _normalize_body(g_ref, d_ref, go_ref, do_ref):
    x = g_ref[...]
    go_ref[...] = x / jnp.sqrt(jnp.sum(x * x, axis=1, keepdims=True))
    y = d_ref[...]
    do_ref[...] = y / jnp.sqrt(jnp.sum(y * y, axis=1, keepdims=True))


def _normalize(h_gene, h_disease):
    return pl.pallas_call(
        _normalize_body,
        out_shape=(
            jax.ShapeDtypeStruct((N_GENE, D), jnp.float32),
            jax.ShapeDtypeStruct((N_DISEASE, D), jnp.float32),
        ),
    )(h_gene, h_disease)


@functools.partial(
    pl.kernel,
    mesh=plsc.VectorSubcoreMesh(core_axis_name="c", subcore_axis_name="s",
                                num_cores=NC),
    out_type=jax.ShapeDtypeStruct((E,), jnp.float32),
    compiler_params=pltpu.CompilerParams(needs_layout_passes=False),
    scratch_types=[
        pltpu.VMEM((PW,), jnp.int32),
        pltpu.VMEM((PW,), jnp.int32),
        pltpu.VMEM((G, D), jnp.float32),
        pltpu.VMEM((G, D), jnp.float32),
        pltpu.VMEM((G, D), jnp.float32),
        pltpu.VMEM((G, D), jnp.float32),
        pltpu.VMEM((PW,), jnp.float32),
        pltpu.SemaphoreType.DMA,
        pltpu.SemaphoreType.DMA,
    ],
)
def _edge_dots(hg_hbm, hd_hbm, src_hbm, dst_hbm, out_hbm,
               src_v, dst_v, u0, v0, u1, v1, o_v, s0, s1):
    wid = lax.axis_index("s") * NC + lax.axis_index("c")
    base = pl.multiple_of(wid * PW, 8)

    pltpu.sync_copy(src_hbm.at[pl.ds(base, PW)], src_v)
    pltpu.sync_copy(dst_hbm.at[pl.ds(base, PW)], dst_v)

    iota16 = lax.iota(jnp.int32, 16)

    def issue(c, ub, vb, sem):
        off = pl.multiple_of(c * G, 8)
        pltpu.async_copy(hg_hbm.at[src_v.at[pl.ds(off, G)]], ub, sem)
        pltpu.async_copy(hd_hbm.at[dst_v.at[pl.ds(off, G)]], vb, sem)

    def wait2(ub, vb, sem):
        pltpu.make_async_copy(hg_hbm.at[src_v.at[pl.ds(0, G)]], ub, sem).wait()
        pltpu.make_async_copy(hd_hbm.at[dst_v.at[pl.ds(0, G)]], vb, sem).wait()

    def compute(c, ub, vb):
        off = pl.multiple_of(c * G, 8)
        zero = jnp.zeros((16,), jnp.float32)
        for g in range(G // 16):
            rows = iota16 + (g * 16)

            def d_body(i, accs):
                a0, a1 = accs
                col0 = (iota16 + 2 * i) & (D - 1)
                col1 = (iota16 + 2 * i + 1) & (D - 1)
                a0 = a0 + plsc.load_gather(ub, [rows, col0]) * \
                    plsc.load_gather(vb, [rows, col0])
                a1 = a1 + plsc.load_gather(ub, [rows, col1]) * \
                    plsc.load_gather(vb, [rows, col1])
                return (a0, a1)

            a0, a1 = lax.fori_loop(0, D // 2, d_body, (zero, zero), unroll=8)
            o_v[pl.ds(off + g * 16, 16)] = a0 + a1

    issue(0, u0, v0, s0)

    def pair_body(t, _):
        c0 = 2 * t
        issue(c0 + 1, u1, v1, s1)
        wait2(u0, v0, s0)
        compute(c0, u0, v0)
        issue(c0 + 2, u0, v0, s0)
        wait2(u1, v1, s1)
        compute(c0 + 1, u1, v1)
        return 0

    lax.fori_loop(0, (NCHUNK - 1) // 2, pair_body, 0)
    wait2(u0, v0, s0)
    compute(NCHUNK - 1, u0, v0)
    pltpu.sync_copy(o_v, out_hbm.at[pl.ds(base, PW)])


def kernel(h_gene, h_disease, edge_index):
    gn, dn = _normalize(h_gene, h_disease)
    src = edge_index[0].astype(jnp.int32)
    dst = edge_index[1].astype(jnp.int32)
    out = _edge_dots(gn, dn, src, dst)
    return out.reshape(E, 1)
